# Initial kernel scaffold; baseline (speedup 1.0000x reference)
import jax
import jax.numpy as jnp
from jax.experimental import pallas as pl


def kernel(x, edge_index, W1l, b1l, W1r, W2l, b2l, W2r):
    raise NotImplementedError("write your pallas kernel here")


# speedup vs baseline: 4.5228x; 4.5228x over previous
import functools

import jax
import jax.numpy as jnp
from jax import lax
from jax.experimental import pallas as pl
from jax.experimental.pallas import tpu as pltpu
from jax.experimental.pallas import tpu_sc as plsc

NC = 2
NS = 16
NW = NC * NS
CHUNK = 128
BLK = 256


def _make_agg(n_pad, d, cpw, with_deg):
    mesh = plsc.VectorSubcoreMesh(core_axis_name="c", subcore_axis_name="s")
    rpt = n_pad // NS

    out_type = [jax.ShapeDtypeStruct((NC * n_pad, d), jnp.float32)]
    scratch = [
        pltpu.VMEM((CHUNK,), jnp.int32),
        pltpu.VMEM((CHUNK,), jnp.int32),
        pltpu.VMEM((CHUNK, d), jnp.float32),
        pltpu.VMEM_SHARED((n_pad, d), jnp.float32),
        pltpu.SemaphoreType.DMA,
    ]
    if with_deg:
        out_type.append(jax.ShapeDtypeStruct((NW * n_pad,), jnp.float32))
        scratch.append(pltpu.VMEM((n_pad,), jnp.float32))

    @functools.partial(
        pl.kernel, out_type=tuple(out_type), mesh=mesh,
        scratch_types=tuple(scratch),
        compiler_params=pltpu.CompilerParams(needs_layout_passes=False))
    def agg_kernel(feat_hbm, src_hbm, dst_hbm, z2_hbm, z1_hbm, *rest):
        if with_deg:
            agg_out, deg_out, src_v, dst_v, rows_v, agg_sh, sg, deg_v = rest
        else:
            agg_out, src_v, dst_v, rows_v, agg_sh, sg = rest
        cid = lax.axis_index("c")
        sid = lax.axis_index("s")
        wid = cid * NS + sid
        ebase = wid * cpw * CHUNK

        srpt = pl.multiple_of(sid * rpt, 8)
        pltpu.sync_copy(z2_hbm.at[pl.ds(srpt, rpt)],
                        agg_sh.at[pl.ds(srpt, rpt)])
        if with_deg:
            pltpu.sync_copy(z1_hbm, deg_v)
        plsc.subcore_barrier()

        def step(c):
            off = ebase + c * CHUNK
            pltpu.sync_copy(src_hbm.at[pl.ds(off, CHUNK)], src_v)
            pltpu.sync_copy(dst_hbm.at[pl.ds(off, CHUNK)], dst_v)
            pltpu.async_copy(feat_hbm.at[src_v], rows_v, sg).wait()
            pltpu.sync_copy(rows_v, agg_sh.at[dst_v], add=True)
            if with_deg:
                for i in range(CHUNK // 16):
                    dv = dst_v[pl.ds(i * 16, 16)]
                    cnt, last = plsc.scan_count(dv)
                    plsc.addupdate_scatter(
                        deg_v, [dv], cnt.astype(jnp.float32), mask=last)

        pl.loop(0, cpw)(step)

        plsc.subcore_barrier()
        obase = pl.multiple_of(cid * n_pad + sid * rpt, 8)
        pltpu.sync_copy(agg_sh.at[pl.ds(srpt, rpt)],
                        agg_out.at[pl.ds(obase, rpt)])
        if with_deg:
            pltpu.sync_copy(deg_v, deg_out.at[pl.ds(wid * n_pad, n_pad)])

    return agg_kernel


def _tc1_body(a_ref, dg_ref, x_ref, w1l_ref, b1l_ref, w1r_ref,
              w2lr_ref, pq_ref):
    agg = a_ref[0] + a_ref[1]
    deg = jnp.maximum(jnp.sum(dg_ref[...], axis=0), 1.0)
    mean = agg / deg[:, None]
    h = (jnp.dot(mean, w1l_ref[...], preferred_element_type=jnp.float32)
         + b1l_ref[...]
         + jnp.dot(x_ref[...], w1r_ref[...], preferred_element_type=jnp.float32))
    h = jnp.maximum(h, 0.0)
    pq_ref[...] = jnp.dot(h, w2lr_ref[...], preferred_element_type=jnp.float32)


def _tc2_body(a_ref, dg_ref, b2l_ref, pq_ref, o_ref):
    d_out = o_ref.shape[-1]
    agg = (a_ref[0] + a_ref[1])[:, :d_out]
    deg = jnp.maximum(jnp.sum(dg_ref[...], axis=0), 1.0)
    q2 = pq_ref[:, d_out:]
    val = agg / deg[:, None] + b2l_ref[...] + q2
    m = jnp.max(val, axis=1, keepdims=True)
    ex = jnp.exp(val - m)
    s = jnp.sum(ex, axis=1, keepdims=True)
    o_ref[...] = val - m - jnp.log(s)


def kernel(x, edge_index, W1l, b1l, W1r, W2l, b2l, W2r):
    n, d_in = x.shape
    d_h = W1l.shape[1]
    d_out = W2l.shape[1]
    e = edge_index.shape[1]

    n_pad = -(-n // BLK) * BLK
    e_pad = -(-e // (NW * CHUNK)) * (NW * CHUNK)
    cpw = e_pad // (NW * CHUNK)

    src = edge_index[0]
    dst = edge_index[1]
    src_p = jnp.concatenate([src, jnp.zeros((e_pad - e,), jnp.int32)])
    dst_p = jnp.concatenate([dst, jnp.full((e_pad - e,), n, jnp.int32)])
    x_p = jnp.pad(x, ((0, n_pad - n), (0, 0)))
    d_pq = 2 * d_out
    w2lr = jnp.concatenate([W2l, W2r], axis=1)
    z2a = jnp.zeros((n_pad, d_in), jnp.float32)
    z2b = jnp.zeros((n_pad, d_pq), jnp.float32)
    z1 = jnp.zeros((n_pad,), jnp.float32)

    agg1_flat, deg_flat = _make_agg(n_pad, d_in, cpw, True)(
        x_p, src_p, dst_p, z2a, z1)
    agg1 = agg1_flat.reshape(NC, n_pad, d_in)
    deg = deg_flat.reshape(NW, n_pad)

    grid = (n_pad // BLK,)
    pq = pl.pallas_call(
        _tc1_body,
        grid=grid,
        in_specs=[
            pl.BlockSpec((NC, BLK, d_in), lambda i: (0, i, 0)),
            pl.BlockSpec((NW, BLK), lambda i: (0, i)),
            pl.BlockSpec((BLK, d_in), lambda i: (i, 0)),
            pl.BlockSpec((d_in, d_h), lambda i: (0, 0)),
            pl.BlockSpec((1, d_h), lambda i: (0, 0)),
            pl.BlockSpec((d_in, d_h), lambda i: (0, 0)),
            pl.BlockSpec((d_h, d_pq), lambda i: (0, 0)),
        ],
        out_specs=pl.BlockSpec((BLK, d_pq), lambda i: (i, 0)),
        out_shape=jax.ShapeDtypeStruct((n_pad, d_pq), jnp.float32),
    )(agg1, deg, x_p, W1l, b1l.reshape(1, d_h), W1r, w2lr)

    agg2_flat, = _make_agg(n_pad, d_pq, cpw, False)(
        pq, src_p, dst_p, z2b, z1)
    agg2 = agg2_flat.reshape(NC, n_pad, d_pq)

    out_pad = pl.pallas_call(
        _tc2_body,
        grid=grid,
        in_specs=[
            pl.BlockSpec((NC, BLK, d_pq), lambda i: (0, i, 0)),
            pl.BlockSpec((NW, BLK), lambda i: (0, i)),
            pl.BlockSpec((1, d_out), lambda i: (0, 0)),
            pl.BlockSpec((BLK, d_pq), lambda i: (i, 0)),
        ],
        out_specs=pl.BlockSpec((BLK, d_out), lambda i: (i, 0)),
        out_shape=jax.ShapeDtypeStruct((n_pad, d_out), jnp.float32),
    )(agg2, deg, b2l.reshape(1, d_out), pq)

    return out_pad[:n]
